# Initial kernel scaffold; baseline (speedup 1.0000x reference)
import jax
import jax.numpy as jnp
from jax.experimental import pallas as pl


def kernel(xyz, params):
    raise NotImplementedError("write your pallas kernel here")


# speedup vs baseline: 3.7728x; 3.7728x over previous
import functools

import jax
import jax.numpy as jnp
from jax import lax
from jax.experimental import pallas as pl
from jax.experimental.pallas import tpu as pltpu
from jax.experimental.pallas import tpu_sc as plsc

_INTERPRET = False

B = 8
K = 32


def _fps_call(xyzT, npoint):
    _, _, N = xyzT.shape

    def body(x_ref, out_ref):
        b = pl.program_id(0)
        xr = x_ref[0, 0:1, :]
        yr = x_ref[0, 1:2, :]
        zr = x_ref[0, 2:3, :]
        iota = lax.broadcasted_iota(jnp.int32, (1, N), 1)
        neg = jnp.float32(-1e30)

        def step(i, carry):
            distance, far = carry
            out_ref[b, i] = far
            sel = iota == far
            cx = jnp.max(jnp.where(sel, xr, neg))
            cy = jnp.max(jnp.where(sel, yr, neg))
            cz = jnp.max(jnp.where(sel, zr, neg))
            dx = xr - cx
            dy = yr - cy
            dz = zr - cz
            dist = (dx * dx + dy * dy) + dz * dz
            distance = jnp.minimum(distance, dist)
            m = jnp.max(distance)
            far = jnp.min(jnp.where(distance == m, iota, N)).astype(jnp.int32)
            return distance, far

        init = (jnp.full((1, N), 1e10, jnp.float32), jnp.int32(0))
        lax.fori_loop(0, npoint, step, init)

    return pl.pallas_call(
        body,
        grid=(B,),
        in_specs=[pl.BlockSpec((1, 3, N), lambda b: (b, 0, 0))],
        out_specs=pl.BlockSpec(memory_space=pltpu.SMEM),
        out_shape=jax.ShapeDtypeStruct((B, npoint), jnp.int32),
        interpret=_INTERPRET,
    )(xyzT)


def _ballq_select_call(d, radius):
    _, S, N = d.shape
    SB = min(128, S)
    r2 = float(radius) ** 2

    def body(d_ref, nidx_ref):
        b = pl.program_id(0)
        dv = d_ref[0]
        lane = lax.broadcasted_iota(jnp.int32, (SB, N), 1)
        v = jnp.where(dv <= r2, lane, jnp.int32(N))
        cols = []
        for _ in range(K):
            m = jnp.min(v, axis=1, keepdims=True)
            cols.append(m)
            v = jnp.where(v == m, N, v)
        y = jnp.concatenate(cols, axis=1)
        first = y[:, :1]
        y = jnp.where(y == N, first, y) + b * N
        nidx_ref[0] = y

    return pl.pallas_call(
        body,
        grid=(B, S // SB),
        in_specs=[pl.BlockSpec((1, SB, N), lambda b, t: (b, t, 0))],
        out_specs=pl.BlockSpec((1, SB, K), lambda b, t: (b, t, 0)),
        out_shape=jax.ShapeDtypeStruct((B, S, K), jnp.int32),
        interpret=_INTERPRET,
    )(d)


def _sqdist(src, dst):
    e = jnp.einsum('bnc,bmc->bnm', src, dst)
    ss = src * src
    s1 = (ss[..., 0] + ss[..., 1]) + ss[..., 2]
    dd = dst * dst
    s2 = (dd[..., 0] + dd[..., 1]) + dd[..., 2]
    d = -2.0 * e
    d = d + s1[:, :, None]
    d = d + s2[:, None, :]
    return d


def _knn_call(d):
    _, n, m = d.shape
    TB = min(256, n)

    def body(d_ref, idx_ref, w_ref):
        b = pl.program_id(0)
        d = d_ref[0]
        lane = lax.broadcasted_iota(jnp.int32, (TB, m), 1)
        big = jnp.float32(1e30)
        v = d
        idxs, ds = [], []
        for _ in range(3):
            dmin = jnp.min(v, axis=1, keepdims=True)
            im = jnp.min(jnp.where(v == dmin, lane, m), axis=1, keepdims=True)
            ds.append(dmin)
            idxs.append(im)
            v = jnp.where(lane == im, big, v)
        d3 = jnp.concatenate(ds, axis=1)
        i3 = jnp.concatenate(idxs, axis=1) + b * m
        recip = 1.0 / (d3 + 1e-8)
        w = recip / jnp.sum(recip, axis=1, keepdims=True)
        idx_ref[0] = i3
        w_ref[0] = w

    return pl.pallas_call(
        body,
        grid=(B, n // TB),
        in_specs=[pl.BlockSpec((1, TB, m), lambda b, t: (b, t, 0))],
        out_specs=[
            pl.BlockSpec((1, TB, 3), lambda b, t: (b, t, 0)),
            pl.BlockSpec((1, TB, 3), lambda b, t: (b, t, 0)),
        ],
        out_shape=[
            jax.ShapeDtypeStruct((B, n, 3), jnp.int32),
            jax.ShapeDtypeStruct((B, n, 3), jnp.float32),
        ],
        interpret=_INTERPRET,
    )(d)


def _sc_gather(table, idx):
    V, D = table.shape
    M = idx.shape[0]
    NW = 32
    assert M % NW == 0, M
    bpw = M // NW
    ch = min(128, bpw)
    while bpw % ch or ch % 8:
        ch -= 8
    nch = bpw // ch
    mesh = plsc.VectorSubcoreMesh(core_axis_name="c", subcore_axis_name="s")

    @functools.partial(
        pl.kernel,
        mesh=mesh,
        out_type=jax.ShapeDtypeStruct((M, D), jnp.float32),
        scratch_types=[
            pltpu.VMEM((ch,), jnp.int32),
            pltpu.VMEM((ch, D), jnp.float32),
            pltpu.SemaphoreType.DMA,
        ],
        compiler_params=pltpu.CompilerParams(use_tc_tiling_on_sc=False),
    )
    def k(table_hbm, idx_hbm, out_hbm, idx_v, rows_v, sem):
        wid = lax.axis_index("s") * 2 + lax.axis_index("c")
        base = wid * bpw

        def step(t, carry):
            off = base + t * ch
            pltpu.sync_copy(idx_hbm.at[pl.ds(off, ch)], idx_v)
            pltpu.async_copy(table_hbm.at[idx_v], rows_v, sem).wait()
            pltpu.sync_copy(rows_v, out_hbm.at[pl.ds(off, ch)])
            return carry

        lax.fori_loop(0, nch, step, 0)

    return k(table, idx)


def _matmul_call(inputs, ws, bias, cout, stats, precision=None, sub=None):
    M = inputs[0][0].shape[0] * inputs[0][3]
    BM = min(512, M)
    grid = M // BM
    n_in = len(inputs)

    def body(*refs):
        i = 0
        x_refs, a_refs, c_refs = [], [], []
        for (_, aff, _, _) in inputs:
            x_refs.append(refs[i]); i += 1
            if aff is not None:
                a_refs.append(refs[i]); c_refs.append(refs[i + 1]); i += 2
            else:
                a_refs.append(None); c_refs.append(None)
        sub_ref = None
        if sub is not None:
            sub_ref = refs[i]; i += 1
        w_refs = refs[i:i + n_in]; i += n_in
        b_ref = refs[i]; i += 1
        y_ref = refs[i]; i += 1
        if stats:
            s_ref = refs[i]; q_ref = refs[i + 1]

        acc = jnp.zeros((BM, cout), jnp.float32)
        for j, (xarr, aff, relu, rep) in enumerate(inputs):
            x = x_refs[j][...]
            if j == 0 and sub_ref is not None:
                srep = sub[1]
                ci = x.shape[1]
                s = sub_ref[...]
                s = jnp.broadcast_to(s[:, None, :], (BM // srep, srep, ci))
                x = x - s.reshape(BM, ci)
            if aff is not None:
                x = x * a_refs[j][...] + c_refs[j][...]
            if relu:
                x = jnp.maximum(x, 0.0)
            if rep != 1:
                ci = x.shape[1]
                x = jnp.broadcast_to(x[:, None, :], (BM // rep, rep, ci))
                x = x.reshape(BM, ci)
            acc = acc + jnp.dot(x, w_refs[j][...],
                                preferred_element_type=jnp.float32,
                                precision=precision)
        y = acc + b_ref[...]
        y_ref[...] = y
        if stats:
            pid = pl.program_id(0)

            @pl.when(pid == 0)
            def _init():
                s_ref[...] = jnp.zeros_like(s_ref)
                q_ref[...] = jnp.zeros_like(q_ref)

            s_ref[...] += jnp.sum(y, axis=0, keepdims=True)
            q_ref[...] += jnp.sum(y * y, axis=0, keepdims=True)

    in_specs, args = [], []
    for (xarr, aff, _, rep) in inputs:
        ci = xarr.shape[1]
        bm = BM // rep
        in_specs.append(pl.BlockSpec((bm, ci), lambda i: (i, 0)))
        args.append(xarr)
        if aff is not None:
            for arr in aff:
                in_specs.append(pl.BlockSpec((1, ci), lambda i: (0, 0)))
                args.append(arr)
    if sub is not None:
        sarr, srep = sub
        in_specs.append(pl.BlockSpec((BM // srep, sarr.shape[1]),
                                     lambda i: (i, 0)))
        args.append(sarr)
    for w in ws:
        in_specs.append(pl.BlockSpec(w.shape, lambda i: (0, 0)))
        args.append(w)
    in_specs.append(pl.BlockSpec((1, cout), lambda i: (0, 0)))
    args.append(bias)

    out_specs = [pl.BlockSpec((BM, cout), lambda i: (i, 0))]
    out_shape = [jax.ShapeDtypeStruct((M, cout), jnp.float32)]
    if stats:
        for _ in range(2):
            out_specs.append(pl.BlockSpec((1, cout), lambda i: (0, 0)))
            out_shape.append(jax.ShapeDtypeStruct((1, cout), jnp.float32))

    res = pl.pallas_call(
        body,
        grid=(grid,),
        in_specs=in_specs,
        out_specs=out_specs,
        out_shape=out_shape,
        interpret=_INTERPRET,
    )(*args)
    return res if stats else res[0]


def _affine(colsum, colsumsq, mtot, gamma, beta):
    mean = colsum / mtot
    var = colsumsq / mtot - mean * mean
    a = gamma[None, :] / jnp.sqrt(var + 1e-5)
    c = beta[None, :] - mean * a
    return a, c


def _maxpool_call(y, a, c):
    M, C = y.shape
    BM = min(512, M)
    SB = BM // K

    def body(y_ref, a_ref, c_ref, o_ref):
        x = jnp.maximum(y_ref[...] * a_ref[...] + c_ref[...], 0.0)
        x = x.reshape(SB, K, C)
        o_ref[...] = jnp.max(x, axis=1)

    return pl.pallas_call(
        body,
        grid=(M // BM,),
        in_specs=[
            pl.BlockSpec((BM, C), lambda i: (i, 0)),
            pl.BlockSpec((1, C), lambda i: (0, 0)),
            pl.BlockSpec((1, C), lambda i: (0, 0)),
        ],
        out_specs=pl.BlockSpec((SB, C), lambda i: (i, 0)),
        out_shape=jax.ShapeDtypeStruct((M // K, C), jnp.float32),
        interpret=_INTERPRET,
    )(y, a, c)


def _combine_call(g, w, a, c):
    _, n, _, C = g.shape
    TB = min(256, n)

    def body(g_ref, w_ref, a_ref, c_ref, o_ref):
        av = a_ref[...]
        cv = c_ref[...]
        acc = None
        for k in range(3):
            x = jnp.maximum(g_ref[0, :, k, :] * av + cv, 0.0)
            wk = w_ref[0, :, k:k + 1]
            t = x * wk
            acc = t if acc is None else acc + t
        o_ref[...] = acc

    nt = n // TB
    return pl.pallas_call(
        body,
        grid=(B, nt),
        in_specs=[
            pl.BlockSpec((1, TB, 3, C), lambda b, t: (b, t, 0, 0)),
            pl.BlockSpec((1, TB, 3), lambda b, t: (b, t, 0)),
            pl.BlockSpec((1, C), lambda b, t: (0, 0)),
            pl.BlockSpec((1, C), lambda b, t: (0, 0)),
        ],
        out_specs=pl.BlockSpec((TB, C), lambda b, t: (b * nt + t, 0)),
        out_shape=jax.ShapeDtypeStruct((B * n, C), jnp.float32),
        interpret=_INTERPRET,
    )(g, w, a, c)


def _head_call(yh, a, c, wc, bc):
    M, C = yh.shape
    NC = wc.shape[1]
    BM = min(512, M)

    def body(y_ref, a_ref, c_ref, w_ref, b_ref, o_ref):
        x = jnp.maximum(y_ref[...] * a_ref[...] + c_ref[...], 0.0)
        z = jnp.dot(x, w_ref[...], preferred_element_type=jnp.float32)
        z = z + b_ref[...]
        m = jnp.max(z, axis=1, keepdims=True)
        e = z - m
        lse = jnp.log(jnp.sum(jnp.exp(e), axis=1, keepdims=True))
        o_ref[...] = e - lse

    return pl.pallas_call(
        body,
        grid=(M // BM,),
        in_specs=[
            pl.BlockSpec((BM, C), lambda i: (i, 0)),
            pl.BlockSpec((1, C), lambda i: (0, 0)),
            pl.BlockSpec((1, C), lambda i: (0, 0)),
            pl.BlockSpec((C, NC), lambda i: (0, 0)),
            pl.BlockSpec((1, NC), lambda i: (0, 0)),
        ],
        out_specs=pl.BlockSpec((BM, NC), lambda i: (i, 0)),
        out_shape=jax.ShapeDtypeStruct((M, NC), jnp.float32),
        interpret=_INTERPRET,
    )(yh, a, c, wc, bc)


def _pad_c(x, mult=16):
    c = x.shape[-1]
    p = (-c) % mult
    if p:
        x = jnp.pad(x, [(0, 0)] * (x.ndim - 1) + [(0, p)])
    return x


def _sa_stage(coords, points, npoint, radius, layers):
    N = coords.shape[1]
    coordsT = jnp.transpose(coords, (0, 2, 1))
    fps_idx = _fps_call(coordsT, npoint)
    bidx = jnp.arange(B)[:, None]
    new_xyz = coords[bidx, fps_idx]
    d = _sqdist(new_xyz, coords)
    nidx = _ballq_select_call(d, radius)

    table = _pad_c(jnp.concatenate([coords, points], axis=-1))
    Dp = table.shape[-1]
    g = _sc_gather(table.reshape(B * N, Dp), nidx.reshape(-1))

    lp = layers[0]
    W1 = lp["W"]
    c1out = W1.shape[0]
    W1p = _pad_c(W1).T
    new_rows = new_xyz.reshape(B * npoint, 3)
    centers = jnp.pad(new_rows, ((0, 0), (0, Dp - 3)))
    mtot = B * npoint * K
    y, s, q = _matmul_call(
        [(g, None, False, 1)], [W1p], lp["b"][None, :], c1out, True,
        sub=(centers, K))
    a, c = _affine(s, q, mtot, lp["gamma"], lp["beta"])
    for lp in layers[1:]:
        cout = lp["W"].shape[0]
        y, s, q = _matmul_call([(y, (a, c), True, 1)], [lp["W"].T],
                               lp["b"][None, :], cout, True)
        a, c = _affine(s, q, mtot, lp["gamma"], lp["beta"])
    out = _maxpool_call(y, a, c)
    return new_xyz, out.reshape(B, npoint, -1)


def _fp_stage(c1, c2, p1, p1_aff, p2, p2_aff, layers):
    n = c1.shape[1]
    m = c2.shape[1]
    C2 = p2.shape[-1]
    idx, w = _knn_call(_sqdist(c1, c2))
    g = _sc_gather(p2.reshape(B * m, C2), idx.reshape(-1))
    if p2_aff is None:
        p2_aff = (jnp.ones((1, C2), jnp.float32), jnp.zeros((1, C2), jnp.float32))
    interp = _combine_call(g.reshape(B, n, 3, C2), w, p2_aff[0], p2_aff[1])

    lp = layers[0]
    cout = lp["W"].shape[0]
    mtot = B * n
    if p1 is not None:
        C1 = p1.shape[-1]
        if p1_aff is None:
            p1_aff = (jnp.ones((1, C1), jnp.float32),
                      jnp.zeros((1, C1), jnp.float32))
        ins = [(p1.reshape(B * n, C1), p1_aff, True, 1),
               (interp, None, False, 1)]
        ws = [lp["W"][:, :C1].T, lp["W"][:, C1:].T]
    else:
        ins = [(interp, None, False, 1)]
        ws = [lp["W"].T]
    y, s, q = _matmul_call(ins, ws, lp["b"][None, :], cout, True)
    a, c = _affine(s, q, mtot, lp["gamma"], lp["beta"])
    for lp in layers[1:]:
        cout = lp["W"].shape[0]
        y, s, q = _matmul_call([(y, (a, c), True, 1)], [lp["W"].T],
                               lp["b"][None, :], cout, True)
        a, c = _affine(s, q, mtot, lp["gamma"], lp["beta"])
    return y, (a, c)


def kernel(xyz, params):
    N = xyz.shape[1]

    l1x, l1p = _sa_stage(xyz, xyz, 1024, 0.1, params["sa1"])
    l2x, l2p = _sa_stage(l1x, l1p, 256, 0.2, params["sa2"])
    l3x, l3p = _sa_stage(l2x, l2p, 64, 0.4, params["sa3"])
    l4x, l4p = _sa_stage(l3x, l3p, 16, 0.8, params["sa4"])

    y3, aff3 = _fp_stage(l3x, l4x, l3p, None, l4p, None, params["fp4"])
    y2, aff2 = _fp_stage(l2x, l3x, l2p, None, y3.reshape(B, 64, -1), aff3,
                         params["fp3"])
    y1, aff1 = _fp_stage(l1x, l2x, l1p, None, y2.reshape(B, 256, -1), aff2,
                         params["fp2"])
    y0, aff0 = _fp_stage(xyz, l1x, None, None, y1.reshape(B, 1024, -1), aff1,
                         params["fp1"])

    hp = params["head1"][0]
    yh, sh, qh = _matmul_call([(y0, aff0, True, 1)], [hp["W"].T],
                              hp["b"][None, :], hp["W"].shape[0], True)
    ah, ch = _affine(sh, qh, B * N, hp["gamma"], hp["beta"])
    out = _head_call(yh, ah, ch, params["conv2"]["W"].T,
                     params["conv2"]["b"][None, :])
    return jnp.transpose(out.reshape(B, N, -1), (0, 2, 1))
